# baseline (speedup 1.0000x reference)
import functools

import jax
import jax.numpy as jnp
from jax import lax
from jax.experimental import pallas as pl
from jax.experimental.pallas import tpu as pltpu
from jax.experimental.pallas import tpu_sc as plsc

NUM_SLOTS = 8192
D_MODEL = 1024
TOPK = 8

_TB = 256
_MC = 2048


def _prep_body(x_ref, mem_ref, xn_ref, rn_ref):
    xb = x_ref[...]
    xn_ref[...] = xb / jnp.maximum(
        jnp.sqrt(jnp.sum(xb * xb, axis=1, keepdims=True)), 1e-12)
    mb = mem_ref[...]
    rn_ref[...] = (1.0 / jnp.maximum(
        jnp.sqrt(jnp.sum(mb * mb, axis=1)), 1e-12))[None, None, :]


def _prep(x2, memory):
    ntok = x2.shape[0]
    grid = (ntok // _TB,)
    return pl.pallas_call(
        _prep_body,
        grid=grid,
        in_specs=[
            pl.BlockSpec((_TB, D_MODEL), lambda i: (i, 0)),
            pl.BlockSpec((_TB, D_MODEL), lambda i: (i, 0)),
        ],
        out_specs=[
            pl.BlockSpec((_TB, D_MODEL), lambda i: (i, 0)),
            pl.BlockSpec((1, 1, _TB), lambda i: (i, 0, 0)),
        ],
        out_shape=[
            jax.ShapeDtypeStruct((ntok, D_MODEL), jnp.float32),
            jax.ShapeDtypeStruct((ntok // _TB, 1, _TB), jnp.float32),
        ],
    )(x2, memory)


def _topk_body(scale_ref, x_ref, mem_ref, rn_ref, out_ref, key_ref,
               msc_ref):
    t = pl.program_id(0)
    c = pl.program_id(1)
    nt = pl.num_programs(0)
    nc = pl.num_programs(1)
    neg_inf = jnp.float32(-jnp.inf)

    @pl.when(t < nt - 1)
    def _():
        mem = mem_ref[pl.ds(c * _MC, _MC), :]
        xn = x_ref[...]
        raw = lax.dot_general(xn, mem, (((1,), (1,)), ((), ())),
                              preferred_element_type=jnp.float32)
        sim = raw * rn_ref[0, pl.ds(c * _MC, _MC)][None, :]
        s32 = lax.bitcast_convert_type(sim, jnp.int32)
        fwd = (lax.broadcasted_iota(jnp.int32, (_TB, _MC), 1)
               + jnp.int32(c * _MC))
        field = jnp.where(s32 >= 0, jnp.int32(NUM_SLOTS - 1) - fwd, fwd)
        kbits = (s32 & jnp.int32(-8192)) | field
        key_ref[t % 2, :, pl.ds(c * _MC, _MC)] = (
            lax.bitcast_convert_type(kbits, jnp.float32))

    @pl.when(t > 0)
    def _():
        buf = (t + 1) % 2
        vals = key_ref[buf]
        m0 = jnp.max(vals, axis=1, keepdims=True)
        vals = jnp.where(vals == m0, neg_inf, vals)
        m1 = jnp.max(vals, axis=1, keepdims=True)
        key_ref[buf] = jnp.where(vals == m1, neg_inf, vals)
        lane = lax.broadcasted_iota(jnp.int32, (_TB, TOPK), 1)
        msc_ref[...] = jnp.where(
            lane == 2 * c, m0,
            jnp.where(lane == 2 * c + 1, m1, msc_ref[...]))

        @pl.when(c == nc - 1)
        def _():
            mk = msc_ref[...]
            u = lax.bitcast_convert_type(mk, jnp.int32)
            field = u & jnp.int32(0x1FFF)
            i = jnp.where(u >= 0, jnp.int32(NUM_SLOTS - 1) - field, field)
            e = jnp.exp(mk - mk[:, 0:1])
            w = e / jnp.sum(e, axis=1, keepdims=True) * scale_ref[0]
            out_ref[...] = jnp.concatenate(
                [i, lax.bitcast_convert_type(w, jnp.int32)], axis=1)


def _sim_topk(xn, memory, rn, retrieval_scale):
    ntok = xn.shape[0]
    nblk = ntok // _TB
    grid = (nblk + 1, NUM_SLOTS // _MC)
    return pl.pallas_call(
        _topk_body,
        grid=grid,
        in_specs=[
            pl.BlockSpec(memory_space=pltpu.SMEM),
            pl.BlockSpec((_TB, D_MODEL),
                         lambda t, c: (jnp.minimum(t, nblk - 1), 0)),
            pl.BlockSpec((NUM_SLOTS, D_MODEL), lambda t, c: (0, 0)),
            pl.BlockSpec((1, NUM_SLOTS), lambda t, c: (0, 0)),
        ],
        out_specs=pl.BlockSpec((_TB, 2 * TOPK),
                               lambda t, c: (jnp.maximum(t - 1, 0), 0)),
        out_shape=jax.ShapeDtypeStruct((ntok, 2 * TOPK), jnp.int32),
        scratch_shapes=[
            pltpu.VMEM((2, _TB, NUM_SLOTS), jnp.float32),
            pltpu.VMEM((_TB, TOPK), jnp.float32),
        ],
        compiler_params=pltpu.CompilerParams(
            dimension_semantics=("arbitrary", "arbitrary")),
    )(jnp.reshape(retrieval_scale, (1,)), xn, memory, rn)


_SC_C = 4
_SC_NBUF = 2


def _sc_combine(x2, mem_bf, idx_flat, w_flat):
    ntok = x2.shape[0]
    info = plsc.get_sparse_core_info()
    nwork = info.num_cores * info.num_subcores
    per_w = ntok // nwork
    C = _SC_C
    NB = _SC_NBUF
    nchunks = per_w // C
    mesh = plsc.VectorSubcoreMesh(core_axis_name="c", subcore_axis_name="s")

    @functools.partial(
        pl.kernel,
        mesh=mesh,
        out_type=jax.ShapeDtypeStruct((ntok, D_MODEL), jnp.float32),
        scratch_types=[
            pltpu.VMEM((per_w * TOPK,), jnp.int32),
            pltpu.VMEM((per_w * TOPK + 16,), jnp.float32),
            pltpu.VMEM((NB, C * TOPK, D_MODEL), jnp.float32),
            pltpu.VMEM((NB, C, D_MODEL), jnp.float32),
            pltpu.VMEM((NB, C, D_MODEL), jnp.float32),
            pltpu.SemaphoreType.DMA((NB,)),
            pltpu.SemaphoreType.DMA((NB,)),
        ],
    )
    def k(x_hbm, mem_hbm, idx_hbm, w_hbm, out_hbm,
          idx_v, w_v, rows_v, x_v, out_v, gsem, xsem):
        wid = lax.axis_index("s") * info.num_cores + lax.axis_index("c")
        tok_base = wid * per_w
        pltpu.sync_copy(idx_hbm.at[pl.ds(tok_base * TOPK, per_w * TOPK)],
                        idx_v)
        pltpu.sync_copy(w_hbm.at[pl.ds(tok_base * TOPK, per_w * TOPK)],
                        w_v.at[pl.ds(0, per_w * TOPK)])

        def issue(ci, b):
            pltpu.async_copy(
                mem_hbm.at[idx_v.at[pl.ds(ci * C * TOPK, C * TOPK)]],
                rows_v.at[b], gsem.at[b])
            pltpu.async_copy(x_hbm.at[pl.ds(tok_base + ci * C, C)],
                             x_v.at[b], xsem.at[b])

        for b in range(NB):
            issue(b, b)

        def compute(ci, b):
            pltpu.make_async_copy(mem_hbm.at[idx_v.at[pl.ds(0, C * TOPK)]],
                                  rows_v.at[b], gsem.at[b]).wait()
            pltpu.make_async_copy(x_hbm.at[pl.ds(0, C)],
                                  x_v.at[b], xsem.at[b]).wait()
            for t in range(C):
                e = ci * C * TOPK + t * TOPK
                wv0 = w_v[pl.ds(e, 16)]
                wvec = [wv0[kk] for kk in range(TOPK)]

                def col_body(j, _, t=t, b=b, wvec=wvec):
                    acc0 = x_v[b, t, pl.ds(j * 32, 16)]
                    acc1 = x_v[b, t, pl.ds(j * 32 + 16, 16)]
                    for kk in range(TOPK):
                        r0 = rows_v[b, t * TOPK + kk, pl.ds(j * 32, 16)]
                        r1 = rows_v[b, t * TOPK + kk, pl.ds(j * 32 + 16, 16)]
                        acc0 = acc0 + wvec[kk] * r0
                        acc1 = acc1 + wvec[kk] * r1
                    out_v[b, t, pl.ds(j * 32, 16)] = acc0
                    out_v[b, t, pl.ds(j * 32 + 16, 16)] = acc1
                    return 0

                lax.fori_loop(0, D_MODEL // 32, col_body, 0, unroll=2)
            pltpu.sync_copy(out_v.at[b],
                            out_hbm.at[pl.ds(tok_base + ci * C, C)])

        def ring_body(p, _):
            for b in range(NB):
                ci = p * NB + b
                compute(ci, b)

                @pl.when(ci + NB < nchunks)
                def _(ci=ci, b=b):
                    issue(ci + NB, b)
            return 0

        nfull = nchunks // NB
        lax.fori_loop(0, nfull, ring_body, 0)
        for ci in range(nfull * NB, nchunks):
            compute(ci, ci % NB)

    return k(x2, mem_bf, idx_flat, w_flat)


def kernel(x, memory, retrieval_scale):
    B, N, D = x.shape
    x2 = x.reshape(B * N, D)
    xn2, rn = _prep(x2, memory)
    rn = rn.reshape(1, NUM_SLOTS)
    nh = 4
    h = x2.shape[0] // nh
    parts = []
    for p in range(nh):
        idxw = _sim_topk(xn2[p * h:(p + 1) * h], memory, rn,
                         retrieval_scale)
        idx = idxw[:, :TOPK]
        w = lax.bitcast_convert_type(idxw[:, TOPK:], jnp.float32)
        parts.append(_sc_combine(x2[p * h:(p + 1) * h], memory,
                                 idx.reshape(-1), w.reshape(-1)))
    return jnp.concatenate(parts, axis=0).reshape(B, N, D)


# speedup vs baseline: 1.0441x; 1.0441x over previous
import functools

import jax
import jax.numpy as jnp
from jax import lax
from jax.experimental import pallas as pl
from jax.experimental.pallas import tpu as pltpu
from jax.experimental.pallas import tpu_sc as plsc

NUM_SLOTS = 8192
D_MODEL = 1024
TOPK = 8

_TB = 256
_MC = 2048


def _topk_body(scale_ref, x_ref, mem_ref, out_ref, key_ref, rn_ref,
               msc_ref):
    t = pl.program_id(0)
    c = pl.program_id(1)
    nt = pl.num_programs(0)
    nc = pl.num_programs(1)
    neg_inf = jnp.float32(-jnp.inf)

    @pl.when(t == 0)
    def _():
        mem = mem_ref[pl.ds(c * _MC, _MC), :]
        ss = jnp.sum(mem * mem, axis=1)
        rn_ref[0, pl.ds(c * _MC, _MC)] = 1.0 / jnp.maximum(
            jnp.sqrt(ss), 1e-12)

    @pl.when(t < nt - 1)
    def _():
        mem = mem_ref[pl.ds(c * _MC, _MC), :]
        xb = x_ref[...]
        xn = xb / jnp.maximum(
            jnp.sqrt(jnp.sum(xb * xb, axis=1, keepdims=True)), 1e-12)
        raw = lax.dot_general(xn, mem, (((1,), (1,)), ((), ())),
                              preferred_element_type=jnp.float32)
        sim = raw * rn_ref[0, pl.ds(c * _MC, _MC)][None, :]
        s32 = lax.bitcast_convert_type(sim, jnp.int32)
        fwd = (lax.broadcasted_iota(jnp.int32, (_TB, _MC), 1)
               + jnp.int32(c * _MC))
        field = jnp.where(s32 >= 0, jnp.int32(NUM_SLOTS - 1) - fwd, fwd)
        kbits = (s32 & jnp.int32(-8192)) | field
        key_ref[t % 2, :, pl.ds(c * _MC, _MC)] = (
            lax.bitcast_convert_type(kbits, jnp.float32))

    @pl.when(t > 0)
    def _():
        buf = (t + 1) % 2
        vals = key_ref[buf]
        m0 = jnp.max(vals, axis=1, keepdims=True)
        vals = jnp.where(vals == m0, neg_inf, vals)
        m1 = jnp.max(vals, axis=1, keepdims=True)
        key_ref[buf] = jnp.where(vals == m1, neg_inf, vals)
        lane = lax.broadcasted_iota(jnp.int32, (_TB, TOPK), 1)
        msc_ref[...] = jnp.where(
            lane == 2 * c, m0,
            jnp.where(lane == 2 * c + 1, m1, msc_ref[...]))

        @pl.when(c == nc - 1)
        def _():
            mk = msc_ref[...]
            u = lax.bitcast_convert_type(mk, jnp.int32)
            field = u & jnp.int32(0x1FFF)
            i = jnp.where(u >= 0, jnp.int32(NUM_SLOTS - 1) - field, field)
            e = jnp.exp(mk - mk[:, 0:1])
            w = e / jnp.sum(e, axis=1, keepdims=True) * scale_ref[0]
            out_ref[...] = jnp.concatenate(
                [i, lax.bitcast_convert_type(w, jnp.int32)], axis=1)


def _sim_topk(x2, memory, retrieval_scale):
    ntok = x2.shape[0]
    nblk = ntok // _TB
    grid = (nblk + 1, NUM_SLOTS // _MC)
    return pl.pallas_call(
        _topk_body,
        grid=grid,
        in_specs=[
            pl.BlockSpec(memory_space=pltpu.SMEM),
            pl.BlockSpec((_TB, D_MODEL),
                         lambda t, c: (jnp.minimum(t, nblk - 1), 0)),
            pl.BlockSpec((NUM_SLOTS, D_MODEL), lambda t, c: (0, 0)),
        ],
        out_specs=pl.BlockSpec((_TB, 2 * TOPK),
                               lambda t, c: (jnp.maximum(t - 1, 0), 0)),
        out_shape=jax.ShapeDtypeStruct((ntok, 2 * TOPK), jnp.int32),
        scratch_shapes=[
            pltpu.VMEM((2, _TB, NUM_SLOTS), jnp.float32),
            pltpu.VMEM((1, NUM_SLOTS), jnp.float32),
            pltpu.VMEM((_TB, TOPK), jnp.float32),
        ],
        compiler_params=pltpu.CompilerParams(
            dimension_semantics=("arbitrary", "arbitrary")),
    )(jnp.reshape(retrieval_scale, (1,)), x2, memory)


_SC_C = 4
_SC_NBUF = 2


def _sc_combine(x2, mem_bf, idx_flat, w_flat):
    ntok = x2.shape[0]
    info = plsc.get_sparse_core_info()
    nwork = info.num_cores * info.num_subcores
    per_w = ntok // nwork
    C = _SC_C
    NB = _SC_NBUF
    nchunks = per_w // C
    mesh = plsc.VectorSubcoreMesh(core_axis_name="c", subcore_axis_name="s")

    @functools.partial(
        pl.kernel,
        mesh=mesh,
        out_type=jax.ShapeDtypeStruct((ntok, D_MODEL), jnp.float32),
        scratch_types=[
            pltpu.VMEM((per_w * TOPK,), jnp.int32),
            pltpu.VMEM((per_w * TOPK + 16,), jnp.float32),
            pltpu.VMEM((NB, C * TOPK, D_MODEL), jnp.float32),
            pltpu.VMEM((NB, C, D_MODEL), jnp.float32),
            pltpu.VMEM((NB, C, D_MODEL), jnp.float32),
            pltpu.SemaphoreType.DMA((NB,)),
            pltpu.SemaphoreType.DMA((NB,)),
        ],
    )
    def k(x_hbm, mem_hbm, idx_hbm, w_hbm, out_hbm,
          idx_v, w_v, rows_v, x_v, out_v, gsem, xsem):
        wid = lax.axis_index("s") * info.num_cores + lax.axis_index("c")
        tok_base = wid * per_w
        pltpu.sync_copy(idx_hbm.at[pl.ds(tok_base * TOPK, per_w * TOPK)],
                        idx_v)
        pltpu.sync_copy(w_hbm.at[pl.ds(tok_base * TOPK, per_w * TOPK)],
                        w_v.at[pl.ds(0, per_w * TOPK)])

        def issue(ci, b):
            pltpu.async_copy(
                mem_hbm.at[idx_v.at[pl.ds(ci * C * TOPK, C * TOPK)]],
                rows_v.at[b], gsem.at[b])
            pltpu.async_copy(x_hbm.at[pl.ds(tok_base + ci * C, C)],
                             x_v.at[b], xsem.at[b])

        for b in range(NB):
            issue(b, b)

        def compute(ci, b):
            pltpu.make_async_copy(mem_hbm.at[idx_v.at[pl.ds(0, C * TOPK)]],
                                  rows_v.at[b], gsem.at[b]).wait()
            pltpu.make_async_copy(x_hbm.at[pl.ds(0, C)],
                                  x_v.at[b], xsem.at[b]).wait()
            for t in range(C):
                e = ci * C * TOPK + t * TOPK
                wv0 = w_v[pl.ds(e, 16)]
                wvec = [wv0[kk] for kk in range(TOPK)]

                def col_body(j, _, t=t, b=b, wvec=wvec):
                    acc0 = x_v[b, t, pl.ds(j * 32, 16)]
                    acc1 = x_v[b, t, pl.ds(j * 32 + 16, 16)]
                    for kk in range(TOPK):
                        r0 = rows_v[b, t * TOPK + kk, pl.ds(j * 32, 16)]
                        r1 = rows_v[b, t * TOPK + kk, pl.ds(j * 32 + 16, 16)]
                        acc0 = acc0 + wvec[kk] * r0
                        acc1 = acc1 + wvec[kk] * r1
                    out_v[b, t, pl.ds(j * 32, 16)] = acc0
                    out_v[b, t, pl.ds(j * 32 + 16, 16)] = acc1
                    return 0

                lax.fori_loop(0, D_MODEL // 32, col_body, 0, unroll=2)
            pltpu.sync_copy(out_v.at[b],
                            out_hbm.at[pl.ds(tok_base + ci * C, C)])

        def ring_body(p, _):
            for b in range(NB):
                ci = p * NB + b
                compute(ci, b)

                @pl.when(ci + NB < nchunks)
                def _(ci=ci, b=b):
                    issue(ci + NB, b)
            return 0

        nfull = nchunks // NB
        lax.fori_loop(0, nfull, ring_body, 0)
        for ci in range(nfull * NB, nchunks):
            compute(ci, ci % NB)

    return k(x2, mem_bf, idx_flat, w_flat)


def kernel(x, memory, retrieval_scale):
    B, N, D = x.shape
    x2 = x.reshape(B * N, D)
    nh = 4
    h = x2.shape[0] // nh
    parts = []
    for p in range(nh):
        xp = x2[p * h:(p + 1) * h]
        idxw = _sim_topk(xp, memory, retrieval_scale)
        idx = idxw[:, :TOPK]
        w = lax.bitcast_convert_type(idxw[:, TOPK:], jnp.float32)
        parts.append(_sc_combine(xp, memory, idx.reshape(-1),
                                 w.reshape(-1)))
    return jnp.concatenate(parts, axis=0).reshape(B, N, D)
